# baseline (speedup 1.0000x reference)
import jax
import jax.numpy as jnp
from jax.experimental import pallas as pl
from jax.experimental.pallas import tpu as pltpu

_SCALE = 0.707106781
_BH = 8


def _body(h_ref, w_ref, o_hbm, buf, sem):
    i = pl.program_id(0)
    n = pl.num_programs(0)
    s = i % 2
    half = _BH // 2

    @pl.when(i >= 2)
    def _():
        pltpu.make_async_copy(
            buf.at[s, pl.ds(0, half)], o_hbm.at[pl.ds(0, half)], sem.at[s, 0]
        ).wait()
        pltpu.make_async_copy(
            buf.at[s, pl.ds(0, half)], o_hbm.at[pl.ds(0, half)], sem.at[s, 1]
        ).wait()

    hs = h_ref[...] * _SCALE
    ws = w_ref[...] * _SCALE
    buf[s] = hs[:, None, :] + ws[None, :, :]

    pltpu.make_async_copy(
        buf.at[s, pl.ds(0, half)],
        o_hbm.at[pl.ds(i * _BH, half)],
        sem.at[s, 0],
    ).start()
    pltpu.make_async_copy(
        buf.at[s, pl.ds(half, half)],
        o_hbm.at[pl.ds(i * _BH + half, half)],
        sem.at[s, 1],
    ).start()

    @pl.when(i == n - 1)
    def _():
        for ss in range(2):
            pltpu.make_async_copy(
                buf.at[ss, pl.ds(0, half)], o_hbm.at[pl.ds(0, half)], sem.at[ss, 0]
            ).wait()
            pltpu.make_async_copy(
                buf.at[ss, pl.ds(0, half)], o_hbm.at[pl.ds(0, half)], sem.at[ss, 1]
            ).wait()


def kernel(height, width, h_embed, w_embed):
    max_h, dim = h_embed.shape
    max_w = w_embed.shape[0]
    return pl.pallas_call(
        _body,
        grid=(max_h // _BH,),
        in_specs=[
            pl.BlockSpec((_BH, dim), lambda i: (i, 0)),
            pl.BlockSpec((max_w, dim), lambda i: (0, 0)),
        ],
        out_specs=pl.BlockSpec(memory_space=pl.ANY),
        out_shape=jax.ShapeDtypeStruct((max_h, max_w, dim), jnp.float32),
        scratch_shapes=[
            pltpu.VMEM((2, _BH, max_w, dim), jnp.float32),
            pltpu.SemaphoreType.DMA((2, 2)),
        ],
    )(h_embed, w_embed)
